# baseline (speedup 1.0000x reference)
import jax
import jax.numpy as jnp
from jax import lax
from jax.experimental import pallas as pl
from jax.experimental.pallas import tpu as pltpu
from jax.experimental.pallas import tpu_sc as plsc

D = 64
N_TOTAL = 16384 * 20
NW = 32
N_PER_W = N_TOTAL // NW
CHUNK = 512
N_CHUNKS = N_PER_W // CHUNK
GPC = CHUNK // 16
IDX_MINOR = 128
IDX_ROWS_PER_W = N_PER_W // IDX_MINOR
GATHERS_PER_CHUNK = CHUNK // IDX_MINOR


def _sc_body(ci_hbm, xi_hbm, in_emb_hbm, out_emb_hbm, out_hbm,
             ci_v, xi_v, c_rows, x_rows, tmp, out_chunk, sem):
    wid = lax.axis_index("s") * 2 + lax.axis_index("c")
    idx_base = wid * IDX_ROWS_PER_W
    pltpu.sync_copy(ci_hbm.at[pl.ds(idx_base, IDX_ROWS_PER_W)], ci_v)
    pltpu.sync_copy(xi_hbm.at[pl.ds(idx_base, IDX_ROWS_PER_W)], xi_v)

    iota16 = lax.iota(jnp.int32, 16)

    def chunk_body(chunk, carry):
        copies = []
        for j in range(GATHERS_PER_CHUNK):
            copies.append(pltpu.async_copy(
                in_emb_hbm.at[ci_v.at[chunk * GATHERS_PER_CHUNK + j]],
                c_rows.at[pl.ds(j * IDX_MINOR, IDX_MINOR)], sem))
            copies.append(pltpu.async_copy(
                out_emb_hbm.at[xi_v.at[chunk * GATHERS_PER_CHUNK + j]],
                x_rows.at[pl.ds(j * IDX_MINOR, IDX_MINOR)], sem))
        for cp in copies:
            cp.wait()

        def group_body(g, gcarry):
            base = g * 16
            acc = jnp.zeros((16,), jnp.float32)
            for r in range(16):
                row = base + r
                s = (c_rows[row, pl.ds(0, 16)] * x_rows[row, pl.ds(0, 16)]
                     + c_rows[row, pl.ds(16, 16)] * x_rows[row, pl.ds(16, 16)]
                     + c_rows[row, pl.ds(32, 16)] * x_rows[row, pl.ds(32, 16)]
                     + c_rows[row, pl.ds(48, 16)] * x_rows[row, pl.ds(48, 16)])
                acc = jnp.where(iota16 == r, jnp.sum(s), acc)
            out_chunk[pl.ds(base, 16)] = acc
            return gcarry

        lax.fori_loop(0, GPC, group_body, 0)
        pltpu.sync_copy(out_chunk,
                        out_hbm.at[pl.ds(wid * N_PER_W + chunk * CHUNK, CHUNK)])
        return carry

    lax.fori_loop(0, N_CHUNKS, chunk_body, 0)


def kernel(center_words, context_words, in_embeddings, out_embeddings):
    B, L = center_words.shape
    ci = center_words.astype(jnp.int32).reshape(N_TOTAL // IDX_MINOR, IDX_MINOR)
    xi = context_words.astype(jnp.int32).reshape(N_TOTAL // IDX_MINOR, IDX_MINOR)
    mesh = plsc.VectorSubcoreMesh(core_axis_name="c", subcore_axis_name="s")
    scores = pl.kernel(
        _sc_body,
        mesh=mesh,
        compiler_params=pltpu.CompilerParams(
            needs_layout_passes=False, use_tc_tiling_on_sc=False),
        out_type=jax.ShapeDtypeStruct((N_TOTAL,), jnp.float32),
        scratch_types=[
            pltpu.VMEM((IDX_ROWS_PER_W, IDX_MINOR), jnp.int32),
            pltpu.VMEM((IDX_ROWS_PER_W, IDX_MINOR), jnp.int32),
            pltpu.VMEM((CHUNK, D), jnp.float32),
            pltpu.VMEM((CHUNK, D), jnp.float32),
            pltpu.VMEM((256,), jnp.float32),
            pltpu.VMEM((CHUNK,), jnp.float32),
            pltpu.SemaphoreType.DMA,
        ],
    )(ci, xi, in_embeddings, out_embeddings)
    return scores.reshape(B, L)


# speedup vs baseline: 1.0464x; 1.0464x over previous
import jax
import jax.numpy as jnp
from jax import lax
from jax.experimental import pallas as pl
from jax.experimental.pallas import tpu as pltpu
from jax.experimental.pallas import tpu_sc as plsc

D = 64
N_TOTAL = 16384 * 20
NW = 32
N_PER_W = N_TOTAL // NW
CHUNK = 256
N_CHUNKS = N_PER_W // CHUNK
N_PAIRS = N_CHUNKS // 2
GPC = CHUNK // 16
IDX_MINOR = 128
IDX_ROWS_PER_W = N_PER_W // IDX_MINOR
GATHERS_PER_CHUNK = CHUNK // IDX_MINOR


def _sc_body(ci_hbm, xi_hbm, in_emb_hbm, out_emb_hbm, out_hbm,
             ci_v, xi_v, c_rows0, x_rows0, c_rows1, x_rows1,
             ob0, ob1, sem0, sem1, osem):
    wid = lax.axis_index("s") * 2 + lax.axis_index("c")
    idx_base = wid * IDX_ROWS_PER_W
    out_base = wid * N_PER_W
    pltpu.sync_copy(ci_hbm.at[pl.ds(idx_base, IDX_ROWS_PER_W)], ci_v)
    pltpu.sync_copy(xi_hbm.at[pl.ds(idx_base, IDX_ROWS_PER_W)], xi_v)

    iota16 = lax.iota(jnp.int32, 16)

    def fire(c, cbuf, xbuf, sem):
        for j in range(GATHERS_PER_CHUNK):
            pltpu.async_copy(in_emb_hbm.at[ci_v.at[c * GATHERS_PER_CHUNK + j]],
                             cbuf.at[pl.ds(j * IDX_MINOR, IDX_MINOR)], sem)
            pltpu.async_copy(out_emb_hbm.at[xi_v.at[c * GATHERS_PER_CHUNK + j]],
                             xbuf.at[pl.ds(j * IDX_MINOR, IDX_MINOR)], sem)

    def drain(c, cbuf, xbuf, sem):
        for j in range(GATHERS_PER_CHUNK):
            pltpu.make_async_copy(
                in_emb_hbm.at[ci_v.at[c * GATHERS_PER_CHUNK + j]],
                cbuf.at[pl.ds(j * IDX_MINOR, IDX_MINOR)], sem).wait()
            pltpu.make_async_copy(
                out_emb_hbm.at[xi_v.at[c * GATHERS_PER_CHUNK + j]],
                xbuf.at[pl.ds(j * IDX_MINOR, IDX_MINOR)], sem).wait()

    def compute(c, cbuf, xbuf, obuf):
        def group_body(g, gcarry):
            base = g * 16
            acc = jnp.zeros((16,), jnp.float32)
            for r in range(16):
                row = base + r
                s = (cbuf[row, pl.ds(0, 16)] * xbuf[row, pl.ds(0, 16)]
                     + cbuf[row, pl.ds(16, 16)] * xbuf[row, pl.ds(16, 16)]
                     + cbuf[row, pl.ds(32, 16)] * xbuf[row, pl.ds(32, 16)]
                     + cbuf[row, pl.ds(48, 16)] * xbuf[row, pl.ds(48, 16)])
                acc = jnp.where(iota16 == r, jnp.sum(s), acc)
            obuf[pl.ds(base, 16)] = acc
            return gcarry

        lax.fori_loop(0, GPC, group_body, 0)
        pltpu.async_copy(obuf, out_hbm.at[pl.ds(out_base + c * CHUNK, CHUNK)],
                         osem)

    def drain_out(c, obuf):
        pltpu.make_async_copy(
            obuf, out_hbm.at[pl.ds(out_base + c * CHUNK, CHUNK)], osem).wait()

    fire(0, c_rows0, x_rows0, sem0)

    def pair_body(i, carry):
        c0 = 2 * i
        c1 = c0 + 1
        fire(c1, c_rows1, x_rows1, sem1)
        drain(c0, c_rows0, x_rows0, sem0)

        @pl.when(i > 0)
        def _():
            drain_out(c0 - 2, ob0)
        compute(c0, c_rows0, x_rows0, ob0)

        @pl.when(i < N_PAIRS - 1)
        def _():
            fire(c0 + 2, c_rows0, x_rows0, sem0)
        drain(c1, c_rows1, x_rows1, sem1)

        @pl.when(i > 0)
        def _():
            drain_out(c1 - 2, ob1)
        compute(c1, c_rows1, x_rows1, ob1)
        return carry

    lax.fori_loop(0, N_PAIRS, pair_body, 0)
    drain_out(N_CHUNKS - 2, ob0)
    drain_out(N_CHUNKS - 1, ob1)


def kernel(center_words, context_words, in_embeddings, out_embeddings):
    B, L = center_words.shape
    ci = center_words.astype(jnp.int32).reshape(N_TOTAL // IDX_MINOR, IDX_MINOR)
    xi = context_words.astype(jnp.int32).reshape(N_TOTAL // IDX_MINOR, IDX_MINOR)
    mesh = plsc.VectorSubcoreMesh(core_axis_name="c", subcore_axis_name="s")
    scores = pl.kernel(
        _sc_body,
        mesh=mesh,
        compiler_params=pltpu.CompilerParams(
            needs_layout_passes=False, use_tc_tiling_on_sc=False),
        out_type=jax.ShapeDtypeStruct((N_TOTAL,), jnp.float32),
        scratch_types=[
            pltpu.VMEM((IDX_ROWS_PER_W, IDX_MINOR), jnp.int32),
            pltpu.VMEM((IDX_ROWS_PER_W, IDX_MINOR), jnp.int32),
            pltpu.VMEM((CHUNK, D), jnp.float32),
            pltpu.VMEM((CHUNK, D), jnp.float32),
            pltpu.VMEM((CHUNK, D), jnp.float32),
            pltpu.VMEM((CHUNK, D), jnp.float32),
            pltpu.VMEM((CHUNK,), jnp.float32),
            pltpu.VMEM((CHUNK,), jnp.float32),
            pltpu.SemaphoreType.DMA,
            pltpu.SemaphoreType.DMA,
            pltpu.SemaphoreType.DMA,
        ],
    )(ci, xi, in_embeddings, out_embeddings)
    return scores.reshape(B, L)
